# baseline (speedup 1.0000x reference)
import jax
import jax.numpy as jnp
from jax.experimental import pallas as pl

_N = 10000
_E = 320000
_D = 128
_H = 8
_DH = 16
_HOP = 5
_ALPHA = 0.15
_DFF = 512
_BN = 400


def _proj_body(feat_ref, g_ref, b_ref, wh_ref, wt_ref, we_ref,
               fh_ref, ft_ref, fe_ref):
    x = feat_ref[...]
    mu = jnp.mean(x, axis=-1, keepdims=True)
    var = jnp.mean((x - mu) * (x - mu), axis=-1, keepdims=True)
    xn = (x - mu) * jax.lax.rsqrt(var + 1e-5) * g_ref[...] + b_ref[...]
    fh_ref[...] = jnp.dot(xn, wh_ref[...], preferred_element_type=jnp.float32)
    ft_ref[...] = jnp.dot(xn, wt_ref[...], preferred_element_type=jnp.float32)
    fe_ref[...] = jnp.dot(xn, we_ref[...], preferred_element_type=jnp.float32)


def _ffn_body(rst_ref, g_ref, b_ref, w1_ref, b1_ref, w2_ref, b2_ref, out_ref):
    r = rst_ref[...]
    mu = jnp.mean(r, axis=-1, keepdims=True)
    var = jnp.mean((r - mu) * (r - mu), axis=-1, keepdims=True)
    y = (r - mu) * jax.lax.rsqrt(var + 1e-5) * g_ref[...] + b_ref[...]
    h = jnp.maximum(
        jnp.dot(y, w1_ref[...], preferred_element_type=jnp.float32)
        + b1_ref[...], 0.0)
    out_ref[...] = (
        jnp.dot(h, w2_ref[...], preferred_element_type=jnp.float32)
        + b2_ref[...] + r)


def kernel(feat, edge_index, ln1_g, ln1_b, W_head, W_tail, W_ent, attn,
           g_head, g_tail, ln2_g, ln2_b, W_ff1, b_ff1, W_ff2, b_ff2):
    row = lambda i: (i, 0)
    full = lambda i: (0, 0)
    grid = (_N // _BN,)

    fh, ft, fe = pl.pallas_call(
        _proj_body,
        grid=grid,
        in_specs=[
            pl.BlockSpec((_BN, _D), row),
            pl.BlockSpec((1, _D), full),
            pl.BlockSpec((1, _D), full),
            pl.BlockSpec((_D, _D), full),
            pl.BlockSpec((_D, _D), full),
            pl.BlockSpec((_D, _D), full),
        ],
        out_specs=[
            pl.BlockSpec((_BN, _D), row),
            pl.BlockSpec((_BN, _D), row),
            pl.BlockSpec((_BN, _D), row),
        ],
        out_shape=[jax.ShapeDtypeStruct((_N, _D), jnp.float32)] * 3,
    )(feat, ln1_g.reshape(1, _D), ln1_b.reshape(1, _D),
      W_head.T, W_tail.T, W_ent.T)

    src = edge_index[0]
    dst = edge_index[1]
    fh3 = fh.reshape(_N, _H, _DH)
    ft3 = ft.reshape(_N, _H, _DH)
    fe3 = fe.reshape(_N, _H, _DH)

    gh = (fh3 * g_head).sum(axis=-1)
    gt = (ft3 * g_tail).sum(axis=-1)
    fta = ft3 * (attn / _DH)

    e = (fh3[src] * fta[dst]).sum(axis=-1)
    in_deg = jax.ops.segment_sum(jnp.ones((_E,), jnp.float32), dst,
                                 num_segments=_N)
    log_in = jnp.log(jnp.maximum(in_deg, 1.0))
    attn_score = e * log_in[dst][:, None]
    gate = jax.nn.sigmoid(gh[src] + gt[dst])

    max_a = jax.ops.segment_max(attn_score, dst, num_segments=_N)
    tag = jnp.exp(attn_score - max_a[dst]) * gate
    sum_ag = jax.ops.segment_sum(tag, dst, num_segments=_N)
    a = (tag / sum_ag[dst])[:, :, None]

    f = fe3
    for _ in range(_HOP):
        m = f[src] * a
        h = jax.ops.segment_sum(m, dst, num_segments=_N)
        f = (1.0 - _ALPHA) * h + _ALPHA * fe3

    rst = (f + feat.reshape(_N, _H, _DH)).reshape(_N, _D)

    out = pl.pallas_call(
        _ffn_body,
        grid=grid,
        in_specs=[
            pl.BlockSpec((_BN, _D), row),
            pl.BlockSpec((1, _D), full),
            pl.BlockSpec((1, _D), full),
            pl.BlockSpec((_D, _DFF), full),
            pl.BlockSpec((1, _DFF), full),
            pl.BlockSpec((_DFF, _D), full),
            pl.BlockSpec((1, _D), full),
        ],
        out_specs=pl.BlockSpec((_BN, _D), row),
        out_shape=jax.ShapeDtypeStruct((_N, _D), jnp.float32),
    )(rst, ln2_g.reshape(1, _D), ln2_b.reshape(1, _D),
      W_ff1.T, b_ff1.reshape(1, _DFF), W_ff2.T, b_ff2.reshape(1, _D))

    return out
